# baseline (speedup 1.0000x reference)
import jax
import jax.numpy as jnp
from jax.experimental import pallas as pl

N = 100000
D_NODE = 128
WARM_DIM = 64
BLOCK = 5000
NUM_BLOCKS = N // BLOCK
NCHUNK = -(-BLOCK // 128)
LB = NCHUNK * 128


def _fused_body(t_ref, x_ref, wmu_ref, bmu_ref, wlv_ref, blv_ref,
                wdec_ref, bdec_ref, out_ref, kl_ref):
    i = pl.program_id(0)
    x = x_ref[...]

    lane = jax.lax.broadcasted_iota(jnp.int32, (128, 128), 1)
    sub = jax.lax.broadcasted_iota(jnp.int32, (128, 128), 0)
    eye = (lane == sub).astype(jnp.float32)
    ones_row = jnp.ones((1, D_NODE), jnp.float32)

    warm_lane = (t_ref[0] == 1).astype(jnp.float32)
    pieces = []
    for c in range(NCHUNK):
        chunk = warm_lane[:, c * 128:(c + 1) * 128]
        col = jax.lax.dot_general(
            eye, chunk, (((1,), (1,)), ((), ())),
            preferred_element_type=jnp.float32)
        pieces.append(jax.lax.dot_general(
            col, ones_row, (((1,), (0,)), ((), ())),
            preferred_element_type=jnp.float32))
    warm_mat = jnp.concatenate(pieces, axis=0)[:BLOCK, :]

    mu = jnp.dot(x, wmu_ref[...], preferred_element_type=jnp.float32) + bmu_ref[...]
    logvar = jnp.dot(x, wlv_ref[...], preferred_element_type=jnp.float32) + blv_ref[...]
    dec = jnp.dot(mu, wdec_ref[...], preferred_element_type=jnp.float32) + bdec_ref[...]

    out_ref[...] = x + warm_mat * (dec - x)

    kl_terms = 1.0 + logvar - mu * mu - jnp.exp(logvar)
    partial = jnp.sum(warm_mat[:, :WARM_DIM] * kl_terms)
    cnt = jnp.sum(warm_mat) * (1.0 / D_NODE)

    lane1 = jax.lax.broadcasted_iota(jnp.int32, (1, 128), 1)
    row = jnp.where(lane1 == 0, partial, 0.0) + jnp.where(lane1 == 1, cnt, 0.0)

    @pl.when(i == 0)
    def _init():
        kl_ref[...] = row

    @pl.when(i > 0)
    def _acc():
        kl_ref[...] += row


def kernel(node_features, node_tiers, W_mu, b_mu, W_logvar, b_logvar, W_dec, b_dec):
    t2 = node_tiers.astype(jnp.int32).reshape(NUM_BLOCKS, BLOCK)
    t3 = jnp.pad(t2, ((0, 0), (0, LB - BLOCK))).reshape(NUM_BLOCKS, 1, LB)

    grid = (NUM_BLOCKS,)
    out_shapes = (
        jax.ShapeDtypeStruct((N, D_NODE), jnp.float32),
        jax.ShapeDtypeStruct((1, 128), jnp.float32),
    )
    new_features, kl_stats = pl.pallas_call(
        _fused_body,
        grid=grid,
        in_specs=[
            pl.BlockSpec((1, 1, LB), lambda i: (i, 0, 0)),
            pl.BlockSpec((BLOCK, D_NODE), lambda i: (i, 0)),
            pl.BlockSpec((D_NODE, WARM_DIM), lambda i: (0, 0)),
            pl.BlockSpec((WARM_DIM,), lambda i: (0,)),
            pl.BlockSpec((D_NODE, WARM_DIM), lambda i: (0, 0)),
            pl.BlockSpec((WARM_DIM,), lambda i: (0,)),
            pl.BlockSpec((WARM_DIM, D_NODE), lambda i: (0, 0)),
            pl.BlockSpec((D_NODE,), lambda i: (0,)),
        ],
        out_specs=(
            pl.BlockSpec((BLOCK, D_NODE), lambda i: (i, 0)),
            pl.BlockSpec((1, 128), lambda i: (0, 0)),
        ),
        out_shape=out_shapes,
    )(t3, node_features, W_mu, b_mu, W_logvar, b_logvar, W_dec, b_dec)

    kl_sum = kl_stats[0, 0]
    n_warm_elems = kl_stats[0, 1] * WARM_DIM
    kl_loss = -0.5 * (kl_sum / n_warm_elems)
    return new_features, kl_loss


# speedup vs baseline: 1.1177x; 1.1177x over previous
import jax
import jax.numpy as jnp
from jax.experimental import pallas as pl

N = 100000
D_NODE = 128
WARM_DIM = 64
BLOCK = 10000
NUM_BLOCKS = N // BLOCK
NCHUNK = -(-BLOCK // 128)
LB = NCHUNK * 128


def _fused_body(t_ref, x_ref, wmu_ref, bmu_ref, wlv_ref, blv_ref,
                wdec_ref, bdec_ref, out_ref, kl_ref):
    i = pl.program_id(0)
    x = x_ref[...]

    lane = jax.lax.broadcasted_iota(jnp.int32, (128, 128), 1)
    sub = jax.lax.broadcasted_iota(jnp.int32, (128, 128), 0)
    eye = (lane == sub).astype(jnp.float32)
    ones_row = jnp.ones((1, D_NODE), jnp.float32)

    warm_lane = (t_ref[0] == 1).astype(jnp.float32)
    pieces = []
    for c in range(NCHUNK):
        chunk = warm_lane[:, c * 128:(c + 1) * 128]
        col = jax.lax.dot_general(
            eye, chunk, (((1,), (1,)), ((), ())),
            preferred_element_type=jnp.float32)
        pieces.append(jax.lax.dot_general(
            col, ones_row, (((1,), (0,)), ((), ())),
            preferred_element_type=jnp.float32))
    warm_mat = jnp.concatenate(pieces, axis=0)[:BLOCK, :]

    mu = jnp.dot(x, wmu_ref[...], preferred_element_type=jnp.float32) + bmu_ref[...]
    logvar = jnp.dot(x, wlv_ref[...], preferred_element_type=jnp.float32) + blv_ref[...]
    dec = jnp.dot(mu, wdec_ref[...], preferred_element_type=jnp.float32) + bdec_ref[...]

    out_ref[...] = x + warm_mat * (dec - x)

    kl_terms = 1.0 + logvar - mu * mu - jnp.exp(logvar)
    partial = jnp.sum(warm_mat[:, :WARM_DIM] * kl_terms)
    cnt = jnp.sum(warm_mat) * (1.0 / D_NODE)

    lane1 = jax.lax.broadcasted_iota(jnp.int32, (1, 128), 1)
    row = jnp.where(lane1 == 0, partial, 0.0) + jnp.where(lane1 == 1, cnt, 0.0)

    @pl.when(i == 0)
    def _init():
        kl_ref[...] = row

    @pl.when(i > 0)
    def _acc():
        kl_ref[...] += row


def kernel(node_features, node_tiers, W_mu, b_mu, W_logvar, b_logvar, W_dec, b_dec):
    t2 = node_tiers.astype(jnp.int32).reshape(NUM_BLOCKS, BLOCK)
    t3 = jnp.pad(t2, ((0, 0), (0, LB - BLOCK))).reshape(NUM_BLOCKS, 1, LB)

    grid = (NUM_BLOCKS,)
    out_shapes = (
        jax.ShapeDtypeStruct((N, D_NODE), jnp.float32),
        jax.ShapeDtypeStruct((1, 128), jnp.float32),
    )
    new_features, kl_stats = pl.pallas_call(
        _fused_body,
        grid=grid,
        in_specs=[
            pl.BlockSpec((1, 1, LB), lambda i: (i, 0, 0)),
            pl.BlockSpec((BLOCK, D_NODE), lambda i: (i, 0)),
            pl.BlockSpec((D_NODE, WARM_DIM), lambda i: (0, 0)),
            pl.BlockSpec((WARM_DIM,), lambda i: (0,)),
            pl.BlockSpec((D_NODE, WARM_DIM), lambda i: (0, 0)),
            pl.BlockSpec((WARM_DIM,), lambda i: (0,)),
            pl.BlockSpec((WARM_DIM, D_NODE), lambda i: (0, 0)),
            pl.BlockSpec((D_NODE,), lambda i: (0,)),
        ],
        out_specs=(
            pl.BlockSpec((BLOCK, D_NODE), lambda i: (i, 0)),
            pl.BlockSpec((1, 128), lambda i: (0, 0)),
        ),
        out_shape=out_shapes,
    )(t3, node_features, W_mu, b_mu, W_logvar, b_logvar, W_dec, b_dec)

    kl_sum = kl_stats[0, 0]
    n_warm_elems = kl_stats[0, 1] * WARM_DIM
    kl_loss = -0.5 * (kl_sum / n_warm_elems)
    return new_features, kl_loss
